# baseline (speedup 1.0000x reference)
import dataclasses
import functools

import jax
import jax.numpy as jnp
from jax import lax
from jax.experimental import pallas as pl
from jax.experimental.pallas import tpu as pltpu
from jax.experimental.pallas import tpu_sc as plsc

_NC = 2
_NS = 16
_NW = _NC * _NS

_CHUNK_BAGS = 4
_SUPER_CHUNKS = 16
_MLP_BM = 1024


def _bag_sums_sc(x_flat, table, B, L, D):
  bags_per_w = B // _NW
  chunk_idx = _CHUNK_BAGS * L
  nchunks = bags_per_w // _CHUNK_BAGS
  nsupers = nchunks // _SUPER_CHUNKS
  super_idx = _SUPER_CHUNKS * chunk_idx
  per_w_idx = bags_per_w * L

  mesh = plsc.VectorSubcoreMesh(core_axis_name="c", subcore_axis_name="s")

  @functools.partial(
      pl.kernel,
      out_type=jax.ShapeDtypeStruct((B, D), jnp.float32),
      mesh=mesh,
      compiler_params=dataclasses.replace(
          pltpu.CompilerParams(use_tc_tiling_on_sc=False),
          needs_layout_passes=False),
      scratch_types=[
          pltpu.VMEM((super_idx,), jnp.int32),
          pltpu.VMEM((super_idx,), jnp.int32),
          pltpu.VMEM((chunk_idx, D), jnp.bfloat16),
          pltpu.VMEM((chunk_idx, D), jnp.bfloat16),
          pltpu.VMEM((bags_per_w, D), jnp.float32),
          pltpu.SemaphoreType.DMA,
          pltpu.SemaphoreType.DMA,
          pltpu.SemaphoreType.DMA,
      ],
  )
  def sc_kernel(x_hbm, tab_hbm, out_hbm, idx0, idx1, rows0, rows1, out_v,
                isem, gsem0, gsem1):
    wid = lax.axis_index("s") * _NC + lax.axis_index("c")
    w_base = wid * per_w_idx

    def fire_idx(s, buf):
      pltpu.async_copy(
          x_hbm.at[pl.ds(w_base + s * super_idx, super_idx)], buf, isem)

    def wait_idx(buf):
      pltpu.make_async_copy(x_hbm.at[pl.ds(0, super_idx)], buf, isem).wait()

    def fire_gathers(idx_buf, off, rows_buf, sem):
      pos = 0
      while pos < chunk_idx:
        n = min(128, chunk_idx - pos)
        pltpu.async_copy(
            tab_hbm.at[idx_buf.at[pl.ds(off + pos, n)]],
            rows_buf.at[pl.ds(pos, n)],
            sem)
        pos += n

    def wait_gathers(rows_buf, sem):
      pltpu.make_async_copy(
          tab_hbm.at[pl.ds(0, chunk_idx)], rows_buf, sem).wait()

    zero = jnp.zeros((16,), jnp.float32)

    def compute(rows_buf, out_row0):
      for bag in range(_CHUNK_BAGS):
        base = bag * L

        def body(i, carry, base=base):
          a0, a1 = carry
          row = rows_buf[base + i, :]
          u0, u1 = plsc.unpack(row, format=plsc.PackFormat.INTERLEAVED)
          a0 = a0 + u0
          a1 = a1 + u1
          return (a0, a1)

        a0, a1 = lax.fori_loop(0, L, body, (zero, zero), unroll=8)
        out_v[out_row0 + bag, pl.ds(0, 16)] = a0
        out_v[out_row0 + bag, pl.ds(16, 16)] = a1

    fire_idx(0, idx0)

    @pl.loop(0, nsupers, step=2)
    def _super(s):
      for q in range(2):
        idxq = idx0 if q == 0 else idx1
        other = idx1 if q == 0 else idx0
        s_q = s + q
        wait_idx(idxq)

        @pl.when(s_q + 1 < nsupers)
        def _():
          fire_idx(s_q + 1, other)

        fire_gathers(idxq, 0, rows0, gsem0)

        @pl.loop(0, _SUPER_CHUNKS, step=2)
        def _chunk(kk):
          wait_gathers(rows0, gsem0)
          fire_gathers(idxq, (kk + 1) * chunk_idx, rows1, gsem1)
          out_row = (s_q * _SUPER_CHUNKS + kk) * _CHUNK_BAGS
          compute(rows0, out_row)
          wait_gathers(rows1, gsem1)

          @pl.when(kk + 2 < _SUPER_CHUNKS)
          def _():
            fire_gathers(idxq, (kk + 2) * chunk_idx, rows0, gsem0)

          compute(rows1, out_row + _CHUNK_BAGS)

    pltpu.sync_copy(out_v, out_hbm.at[pl.ds(wid * bags_per_w, bags_per_w)])

  return sc_kernel(x_flat, table)


def _mlp_tc(sums, fc1_W, fc1_b, fc2_W, fc2_b, inv_l):
  B, D = sums.shape
  H = fc1_W.shape[0]
  O = fc2_W.shape[0]
  bm = _MLP_BM

  def body(s_ref, w1_ref, b1_ref, w2_ref, b2_ref, o_ref):
    t = s_ref[...] * inv_l
    h = lax.dot_general(t, w1_ref[...], (((1,), (1,)), ((), ())),
                        preferred_element_type=jnp.float32)
    h = jnp.maximum(h + b1_ref[...], 0.0)
    o = lax.dot_general(h, w2_ref[...], (((1,), (1,)), ((), ())),
                        preferred_element_type=jnp.float32)
    o_ref[...] = o + b2_ref[...]

  return pl.pallas_call(
      body,
      grid=(B // bm,),
      in_specs=[
          pl.BlockSpec((bm, D), lambda i: (i, 0)),
          pl.BlockSpec((H, D), lambda i: (0, 0)),
          pl.BlockSpec((1, H), lambda i: (0, 0)),
          pl.BlockSpec((O, H), lambda i: (0, 0)),
          pl.BlockSpec((1, O), lambda i: (0, 0)),
      ],
      out_specs=pl.BlockSpec((bm, O), lambda i: (i, 0)),
      out_shape=jax.ShapeDtypeStruct((B, O), jnp.float32),
  )(sums, fc1_W, fc1_b.reshape(1, H), fc2_W, fc2_b.reshape(1, O))


def kernel(x, emb_weight, fc1_W, fc1_b, fc2_W, fc2_b):
  B, L = x.shape
  D = emb_weight.shape[1]
  tab16 = emb_weight.astype(jnp.bfloat16)
  sums = _bag_sums_sc(x.reshape(-1), tab16, B, L, D)
  half = D // 2
  perm = [2 * j for j in range(half)] + [2 * j + 1 for j in range(half)]
  fc1_Wp = fc1_W[:, perm]
  return _mlp_tc(sums, fc1_Wp, fc1_b, fc2_W, fc2_b, 1.0 / L)


# speedup vs baseline: 1.0032x; 1.0032x over previous
import dataclasses
import functools

import jax
import jax.numpy as jnp
from jax import lax
from jax.experimental import pallas as pl
from jax.experimental.pallas import tpu as pltpu
from jax.experimental.pallas import tpu_sc as plsc

_NC = 2
_NS = 16
_NW = _NC * _NS

_CHUNK_BAGS = 4
_SUPER_CHUNKS = 16
_MLP_BM = 1024


def _bag_sums_sc(x, table, B, L, D):
  bags_per_w = B // _NW
  chunk_rows = _CHUNK_BAGS * L
  nchunks = bags_per_w // _CHUNK_BAGS
  nsupers = nchunks // _SUPER_CHUNKS
  super_bags = _SUPER_CHUNKS * _CHUNK_BAGS

  mesh = plsc.VectorSubcoreMesh(core_axis_name="c", subcore_axis_name="s")

  @functools.partial(
      pl.kernel,
      out_type=jax.ShapeDtypeStruct((B, D), jnp.float32),
      mesh=mesh,
      compiler_params=dataclasses.replace(
          pltpu.CompilerParams(use_tc_tiling_on_sc=False),
          needs_layout_passes=False),
      scratch_types=[
          pltpu.VMEM((super_bags, L), jnp.int32),
          pltpu.VMEM((super_bags, L), jnp.int32),
          pltpu.VMEM((chunk_rows, D), jnp.bfloat16),
          pltpu.VMEM((chunk_rows, D), jnp.bfloat16),
          pltpu.VMEM((bags_per_w, D), jnp.float32),
          pltpu.SemaphoreType.DMA,
          pltpu.SemaphoreType.DMA,
          pltpu.SemaphoreType.DMA,
      ],
  )
  def sc_kernel(x_hbm, tab_hbm, out_hbm, idx0, idx1, rows0, rows1, out_v,
                isem, gsem0, gsem1):
    wid = lax.axis_index("s") * _NC + lax.axis_index("c")
    w_bag0 = wid * bags_per_w

    def fire_idx(s, buf):
      pltpu.async_copy(
          x_hbm.at[pl.ds(w_bag0 + s * super_bags, super_bags), :], buf, isem)

    def wait_idx(buf):
      pltpu.make_async_copy(
          x_hbm.at[pl.ds(0, super_bags), :], buf, isem).wait()

    def fire_gathers(idx_buf, kk, rows_buf, sem):
      for bag in range(_CHUNK_BAGS):
        row = kk * _CHUNK_BAGS + bag
        pos = 0
        while pos < L:
          n = min(128, L - pos)
          pltpu.async_copy(
              tab_hbm.at[idx_buf.at[row, pl.ds(pos, n)]],
              rows_buf.at[pl.ds(bag * L + pos, n)],
              sem)
          pos += n

    def wait_gathers(rows_buf, sem):
      pltpu.make_async_copy(
          tab_hbm.at[pl.ds(0, chunk_rows)], rows_buf, sem).wait()

    zero = jnp.zeros((16,), jnp.float32)
    npairs = L // 2

    def compute(rows_buf, out_row0):
      for bag in range(_CHUNK_BAGS):
        base = bag * L

        def body(i, carry, base=base):
          a0, a1 = carry
          p = rows_buf[base + 2 * i, :] + rows_buf[base + 2 * i + 1, :]
          u0, u1 = plsc.unpack(p, format=plsc.PackFormat.INTERLEAVED)
          a0 = a0 + u0
          a1 = a1 + u1
          return (a0, a1)

        a0, a1 = lax.fori_loop(0, npairs, body, (zero, zero), unroll=10)
        out_v[out_row0 + bag, pl.ds(0, 16)] = a0
        out_v[out_row0 + bag, pl.ds(16, 16)] = a1

    fire_idx(0, idx0)

    @pl.loop(0, nsupers, step=2)
    def _super(s):
      for q in range(2):
        idxq = idx0 if q == 0 else idx1
        other = idx1 if q == 0 else idx0
        s_q = s + q
        wait_idx(idxq)

        @pl.when(s_q + 1 < nsupers)
        def _():
          fire_idx(s_q + 1, other)

        fire_gathers(idxq, 0, rows0, gsem0)

        @pl.loop(0, _SUPER_CHUNKS, step=2)
        def _chunk(kk):
          wait_gathers(rows0, gsem0)
          fire_gathers(idxq, kk + 1, rows1, gsem1)
          out_row = (s_q * _SUPER_CHUNKS + kk) * _CHUNK_BAGS
          compute(rows0, out_row)
          wait_gathers(rows1, gsem1)

          @pl.when(kk + 2 < _SUPER_CHUNKS)
          def _():
            fire_gathers(idxq, kk + 2, rows0, gsem0)

          compute(rows1, out_row + _CHUNK_BAGS)

    pltpu.sync_copy(out_v, out_hbm.at[pl.ds(w_bag0, bags_per_w)])

  return sc_kernel(x, table)


def _mlp_tc(sums, fc1_W, fc1_b, fc2_W, fc2_b, inv_l):
  B, D = sums.shape
  H = fc1_W.shape[0]
  O = fc2_W.shape[0]
  bm = _MLP_BM

  def body(s_ref, w1_ref, b1_ref, w2_ref, b2_ref, o_ref):
    t = s_ref[...] * inv_l
    h = lax.dot_general(t, w1_ref[...], (((1,), (1,)), ((), ())),
                        preferred_element_type=jnp.float32)
    h = jnp.maximum(h + b1_ref[...], 0.0)
    o = lax.dot_general(h, w2_ref[...], (((1,), (1,)), ((), ())),
                        preferred_element_type=jnp.float32)
    o_ref[...] = o + b2_ref[...]

  return pl.pallas_call(
      body,
      grid=(B // bm,),
      in_specs=[
          pl.BlockSpec((bm, D), lambda i: (i, 0)),
          pl.BlockSpec((H, D), lambda i: (0, 0)),
          pl.BlockSpec((1, H), lambda i: (0, 0)),
          pl.BlockSpec((O, H), lambda i: (0, 0)),
          pl.BlockSpec((1, O), lambda i: (0, 0)),
      ],
      out_specs=pl.BlockSpec((bm, O), lambda i: (i, 0)),
      out_shape=jax.ShapeDtypeStruct((B, O), jnp.float32),
  )(sums, fc1_W, fc1_b.reshape(1, H), fc2_W, fc2_b.reshape(1, O))


def kernel(x, emb_weight, fc1_W, fc1_b, fc2_W, fc2_b):
  B, L = x.shape
  D = emb_weight.shape[1]
  tab16 = emb_weight.astype(jnp.bfloat16)
  sums = _bag_sums_sc(x, tab16, B, L, D)
  half = D // 2
  perm = [2 * j for j in range(half)] + [2 * j + 1 for j in range(half)]
  fc1_Wp = fc1_W[:, perm]
  return _mlp_tc(sums, fc1_Wp, fc1_b, fc2_W, fc2_b, 1.0 / L)


# speedup vs baseline: 1.0993x; 1.0958x over previous
import dataclasses
import functools

import jax
import jax.numpy as jnp
from jax import lax
from jax.experimental import pallas as pl
from jax.experimental.pallas import tpu as pltpu
from jax.experimental.pallas import tpu_sc as plsc

_NC = 2
_NS = 16
_NW = _NC * _NS

_CHUNK_BAGS = 4
_SUPER_CHUNKS = 16
_MLP_BM = 1024


def _bag_sums_sc(x, table, B, L, D):
  bags_per_w = B // _NW
  chunk_rows = _CHUNK_BAGS * L
  nchunks = bags_per_w // _CHUNK_BAGS
  nsupers = nchunks // _SUPER_CHUNKS
  super_bags = _SUPER_CHUNKS * _CHUNK_BAGS

  mesh = plsc.VectorSubcoreMesh(core_axis_name="c", subcore_axis_name="s")

  @functools.partial(
      pl.kernel,
      out_type=jax.ShapeDtypeStruct((B, D), jnp.float32),
      mesh=mesh,
      compiler_params=dataclasses.replace(
          pltpu.CompilerParams(use_tc_tiling_on_sc=False),
          needs_layout_passes=False),
      scratch_types=[
          pltpu.VMEM((super_bags, L), jnp.int32),
          pltpu.VMEM((super_bags, L), jnp.int32),
          pltpu.VMEM((chunk_rows, D), jnp.float32),
          pltpu.VMEM((chunk_rows, D), jnp.float32),
          pltpu.VMEM((bags_per_w, D), jnp.float32),
          pltpu.SemaphoreType.DMA,
          pltpu.SemaphoreType.DMA,
          pltpu.SemaphoreType.DMA,
      ],
  )
  def sc_kernel(x_hbm, tab_hbm, out_hbm, idx0, idx1, rows0, rows1, out_v,
                isem, gsem0, gsem1):
    wid = lax.axis_index("s") * _NC + lax.axis_index("c")
    w_bag0 = wid * bags_per_w

    def fire_idx(s, buf):
      pltpu.async_copy(
          x_hbm.at[pl.ds(w_bag0 + s * super_bags, super_bags), :], buf, isem)

    def wait_idx(buf):
      pltpu.make_async_copy(
          x_hbm.at[pl.ds(0, super_bags), :], buf, isem).wait()

    def fire_gathers(idx_buf, kk, rows_buf, sem):
      for bag in range(_CHUNK_BAGS):
        row = kk * _CHUNK_BAGS + bag
        pos = 0
        while pos < L:
          n = min(128, L - pos)
          pltpu.async_copy(
              tab_hbm.at[idx_buf.at[row, pl.ds(pos, n)]],
              rows_buf.at[pl.ds(bag * L + pos, n)],
              sem)
          pos += n

    def wait_gathers(rows_buf, sem):
      pltpu.make_async_copy(
          tab_hbm.at[pl.ds(0, chunk_rows)], rows_buf, sem).wait()

    zero = jnp.zeros((16,), jnp.float32)
    npairs = L // 2

    def compute(rows_buf, out_row0):
      for bag in range(_CHUNK_BAGS):
        base = bag * L

        def body(i, carry, base=base):
          a0, a1 = carry
          a0 = a0 + rows_buf[base + i, pl.ds(0, 16)]
          a1 = a1 + rows_buf[base + i, pl.ds(16, 16)]
          return (a0, a1)

        a0, a1 = lax.fori_loop(0, L, body, (zero, zero), unroll=8)
        out_v[out_row0 + bag, pl.ds(0, 16)] = a0
        out_v[out_row0 + bag, pl.ds(16, 16)] = a1

    fire_idx(0, idx0)

    @pl.loop(0, nsupers, step=2)
    def _super(s):
      for q in range(2):
        idxq = idx0 if q == 0 else idx1
        other = idx1 if q == 0 else idx0
        s_q = s + q
        wait_idx(idxq)

        @pl.when(s_q + 1 < nsupers)
        def _():
          fire_idx(s_q + 1, other)

        fire_gathers(idxq, 0, rows0, gsem0)

        @pl.loop(0, _SUPER_CHUNKS, step=2)
        def _chunk(kk):
          wait_gathers(rows0, gsem0)
          fire_gathers(idxq, kk + 1, rows1, gsem1)
          out_row = (s_q * _SUPER_CHUNKS + kk) * _CHUNK_BAGS
          compute(rows0, out_row)
          wait_gathers(rows1, gsem1)

          @pl.when(kk + 2 < _SUPER_CHUNKS)
          def _():
            fire_gathers(idxq, kk + 2, rows0, gsem0)

          compute(rows1, out_row + _CHUNK_BAGS)

    pltpu.sync_copy(out_v, out_hbm.at[pl.ds(w_bag0, bags_per_w)])

  return sc_kernel(x, table)


def _mlp_tc(sums, fc1_W, fc1_b, fc2_W, fc2_b, inv_l):
  B, D = sums.shape
  H = fc1_W.shape[0]
  O = fc2_W.shape[0]
  bm = _MLP_BM

  def body(s_ref, w1_ref, b1_ref, w2_ref, b2_ref, o_ref):
    t = s_ref[...] * inv_l
    h = lax.dot_general(t, w1_ref[...], (((1,), (1,)), ((), ())),
                        preferred_element_type=jnp.float32)
    h = jnp.maximum(h + b1_ref[...], 0.0)
    o = lax.dot_general(h, w2_ref[...], (((1,), (1,)), ((), ())),
                        preferred_element_type=jnp.float32)
    o_ref[...] = o + b2_ref[...]

  return pl.pallas_call(
      body,
      grid=(B // bm,),
      in_specs=[
          pl.BlockSpec((bm, D), lambda i: (i, 0)),
          pl.BlockSpec((H, D), lambda i: (0, 0)),
          pl.BlockSpec((1, H), lambda i: (0, 0)),
          pl.BlockSpec((O, H), lambda i: (0, 0)),
          pl.BlockSpec((1, O), lambda i: (0, 0)),
      ],
      out_specs=pl.BlockSpec((bm, O), lambda i: (i, 0)),
      out_shape=jax.ShapeDtypeStruct((B, O), jnp.float32),
  )(sums, fc1_W, fc1_b.reshape(1, H), fc2_W, fc2_b.reshape(1, O))


def kernel(x, emb_weight, fc1_W, fc1_b, fc2_W, fc2_b):
  B, L = x.shape
  D = emb_weight.shape[1]
  sums = _bag_sums_sc(x, emb_weight, B, L, D)
  return _mlp_tc(sums, fc1_W, fc1_b, fc2_W, fc2_b, 1.0 / L)


# speedup vs baseline: 1.1000x; 1.0007x over previous
import dataclasses
import functools

import jax
import jax.numpy as jnp
from jax import lax
from jax.experimental import pallas as pl
from jax.experimental.pallas import tpu as pltpu
from jax.experimental.pallas import tpu_sc as plsc

_NC = 2
_NS = 16
_NW = _NC * _NS

_CHUNK_BAGS = 4
_SUPER_CHUNKS = 16
_MLP_BM = 1024


def _bag_sums_sc(x, table, B, L, D):
  bags_per_w = B // _NW
  chunk_rows = _CHUNK_BAGS * L
  nchunks = bags_per_w // _CHUNK_BAGS
  nsupers = nchunks // _SUPER_CHUNKS
  super_bags = _SUPER_CHUNKS * _CHUNK_BAGS

  mesh = plsc.VectorSubcoreMesh(core_axis_name="c", subcore_axis_name="s")

  @functools.partial(
      pl.kernel,
      out_type=jax.ShapeDtypeStruct((B, D), jnp.float32),
      mesh=mesh,
      compiler_params=dataclasses.replace(
          pltpu.CompilerParams(use_tc_tiling_on_sc=False),
          needs_layout_passes=False),
      scratch_types=[
          pltpu.VMEM((super_bags, L), jnp.int32),
          pltpu.VMEM((super_bags, L), jnp.int32),
          pltpu.VMEM((chunk_rows, D), jnp.float32),
          pltpu.VMEM((chunk_rows, D), jnp.float32),
          pltpu.VMEM((bags_per_w, D), jnp.float32),
          pltpu.SemaphoreType.DMA,
          pltpu.SemaphoreType.DMA,
          pltpu.SemaphoreType.DMA,
      ],
  )
  def sc_kernel(x_hbm, tab_hbm, out_hbm, idx0, idx1, rows0, rows1, out_v,
                isem, gsem0, gsem1):
    wid = lax.axis_index("s") * _NC + lax.axis_index("c")
    w_bag0 = wid * bags_per_w

    def fire_idx(s, buf):
      pltpu.async_copy(
          x_hbm.at[pl.ds(w_bag0 + s * super_bags, super_bags), :], buf, isem)

    def wait_idx(buf):
      pltpu.make_async_copy(
          x_hbm.at[pl.ds(0, super_bags), :], buf, isem).wait()

    def fire_gathers(idx_buf, kk, rows_buf, sem):
      for bag in range(_CHUNK_BAGS):
        row = kk * _CHUNK_BAGS + bag
        pos = 0
        while pos < L:
          n = min(128, L - pos)
          pltpu.async_copy(
              tab_hbm.at[idx_buf.at[row, pl.ds(pos, n)]],
              rows_buf.at[pl.ds(bag * L + pos, n)],
              sem)
          pos += n

    def wait_gathers(rows_buf, sem):
      pltpu.make_async_copy(
          tab_hbm.at[pl.ds(0, chunk_rows)], rows_buf, sem).wait()

    zero = jnp.zeros((16,), jnp.float32)
    npairs = L // 2

    def compute(rows_buf, out_row0):
      for bag in range(_CHUNK_BAGS):
        base = bag * L

        def body(i, carry, base=base):
          a0, a1 = carry
          a0 = a0 + rows_buf[base + i, pl.ds(0, 16)]
          a1 = a1 + rows_buf[base + i, pl.ds(16, 16)]
          return (a0, a1)

        a0, a1 = lax.fori_loop(0, L, body, (zero, zero), unroll=8)
        out_v[out_row0 + bag, pl.ds(0, 16)] = a0
        out_v[out_row0 + bag, pl.ds(16, 16)] = a1

    fire_idx(0, idx0)

    @pl.loop(0, nsupers, step=2)
    def _super(s):
      for q in range(2):
        idxq = idx0 if q == 0 else idx1
        other = idx1 if q == 0 else idx0
        s_q = s + q
        wait_idx(idxq)

        @pl.when(s_q + 1 < nsupers)
        def _():
          fire_idx(s_q + 1, other)

        fire_gathers(idxq, 0, rows0, gsem0)

        @pl.loop(0, _SUPER_CHUNKS, step=2)
        def _chunk(kk):
          wait_gathers(rows0, gsem0)
          fire_gathers(idxq, kk + 1, rows1, gsem1)
          out_row = (s_q * _SUPER_CHUNKS + kk) * _CHUNK_BAGS
          compute(rows0, out_row)
          wait_gathers(rows1, gsem1)

          @pl.when(kk + 2 < _SUPER_CHUNKS)
          def _():
            fire_gathers(idxq, kk + 2, rows0, gsem0)

          compute(rows1, out_row + _CHUNK_BAGS)

    pltpu.sync_copy(out_v, out_hbm.at[pl.ds(w_bag0, bags_per_w)])

  return sc_kernel(x, table)


def _mlp_tc(sums, fc1_W, fc1_b, fc2_W, fc2_b, inv_l):
  B, D = sums.shape
  H = fc1_W.shape[0]
  O = fc2_W.shape[0]
  bm = _MLP_BM

  def body(s_ref, w1_ref, b1_ref, w2_ref, b2_ref, o_ref):
    t = s_ref[...] * inv_l
    h = lax.dot_general(t, w1_ref[...], (((1,), (1,)), ((), ())),
                        preferred_element_type=jnp.float32)
    h = jnp.maximum(h + b1_ref[...], 0.0)
    o = lax.dot_general(h, w2_ref[...], (((1,), (1,)), ((), ())),
                        preferred_element_type=jnp.float32)
    o_ref[...] = o + b2_ref[...]

  return pl.pallas_call(
      body,
      grid=(B // bm,),
      in_specs=[
          pl.BlockSpec((bm, D), lambda i: (i, 0)),
          pl.BlockSpec((H, D), lambda i: (0, 0)),
          pl.BlockSpec((1, H), lambda i: (0, 0)),
          pl.BlockSpec((O, H), lambda i: (0, 0)),
          pl.BlockSpec((1, O), lambda i: (0, 0)),
      ],
      out_specs=pl.BlockSpec((bm, O), lambda i: (i, 0)),
      out_shape=jax.ShapeDtypeStruct((B, O), jnp.float32),
  )(sums, fc1_W, fc1_b.reshape(1, H), fc2_W, fc2_b.reshape(1, O))


def kernel(x, emb_weight, fc1_W, fc1_b, fc2_W, fc2_b):
  B, L = x.shape
  V, D = emb_weight.shape
  tab_lin = jnp.reshape(
      lax.optimization_barrier(jnp.reshape(emb_weight, (V * D,))), (V, D))
  sums = _bag_sums_sc(x, tab_lin, B, L, D)
  return _mlp_tc(sums, fc1_W, fc1_b, fc2_W, fc2_b, 1.0 / L)
